# baseline (speedup 1.0000x reference)
import jax
import jax.numpy as jnp
from jax import lax
from jax.experimental import pallas as pl
from jax.experimental.pallas import tpu as pltpu
from jax.experimental.pallas import tpu_sc as plsc

N = 10000
E = 160000
D_IN = 256
D_H = 512
D_OUT = 256

NC = 2
NS = 16
GB = 64
IDXB = 8
CHUNKS = -(-E // (NS * GB * IDXB)) * IDXB
NB = CHUNKS // IDXB
EPAD = CHUNKS * NS * GB
RPT = -(-(N + 8) // (NS * 8)) * 8
NPAD = RPT * NS
TAIL = N - (NS - 1) * RPT


def _row_chunks(total):
    out = []
    off = 0
    while off < total:
        sz = min(GB, total - off)
        out.append((off, sz))
        off += sz
    return out


def _sc_agg_body(table, srcx, dstx, zeros_hbm, agg0_out, agg1_out,
                 idx_s, idx_d, gbuf, acc, sem):
    c = lax.axis_index("c")
    s = lax.axis_index("s")
    base = s * RPT

    pltpu.sync_copy(zeros_hbm, gbuf)
    for off, sz in _row_chunks(RPT):
        pltpu.sync_copy(gbuf.at[pl.ds(0, sz)], acc.at[pl.ds(base + off, sz)])
    plsc.subcore_barrier()

    def block(bi, _):
        pltpu.sync_copy(srcx.at[c, s, pl.ds(bi * IDXB, IDXB)], idx_s)
        pltpu.sync_copy(dstx.at[s, pl.ds(bi * IDXB, IDXB)], idx_d)

        def step(j, _):
            pltpu.async_copy(table.at[idx_s.at[j]], gbuf, sem).wait()
            pltpu.sync_copy(gbuf, acc.at[idx_d.at[j]], add=True)
            return _

        return lax.fori_loop(0, IDXB, step, _)

    lax.fori_loop(0, NB, block, None)
    plsc.subcore_barrier()

    for cid, out_ref in ((0, agg0_out), (1, agg1_out)):
        for last in (False, True):
            nrows = TAIL if last else RPT
            cond = jnp.logical_and(c == cid,
                                   (s == NS - 1) if last else (s != NS - 1))
            for off, sz in _row_chunks(nrows):
                @pl.when(cond)
                def _(off=off, sz=sz, out_ref=out_ref):
                    pltpu.sync_copy(acc.at[pl.ds(base + off, sz)],
                                    gbuf.at[pl.ds(0, sz)])
                    pltpu.sync_copy(gbuf.at[pl.ds(0, sz)],
                                    out_ref.at[pl.ds(base + off, sz)])


_SC_MESH = plsc.VectorSubcoreMesh(
    core_axis_name="c", subcore_axis_name="s", num_cores=NC, num_subcores=NS
)

_sc_agg = pl.kernel(
    _sc_agg_body,
    out_type=[
        jax.ShapeDtypeStruct((N, 128), jnp.float32),
        jax.ShapeDtypeStruct((N, 128), jnp.float32),
    ],
    mesh=_SC_MESH,
    scratch_types=[
        pltpu.VMEM((IDXB, GB), jnp.int32),
        pltpu.VMEM((IDXB, GB), jnp.int32),
        pltpu.VMEM((GB, 128), jnp.float32),
        pltpu.VMEM_SHARED((NPAD, 128), jnp.float32),
        pltpu.SemaphoreType.DMA,
    ],
)


def _tc_fused_body(agg0, agg1, x, deg, w1l, w1r, b1, w2l, w2r, b2, y2, z2):
    recip = 1.0 / jnp.maximum(deg[:, :1], 1.0)
    mean = jnp.concatenate([agg0[...] * recip, agg1[...] * recip], axis=1)
    pre = (
        jnp.dot(mean, w1l[...], preferred_element_type=jnp.float32,
                precision=lax.Precision.HIGHEST)
        + jnp.dot(x[...], w1r[...], preferred_element_type=jnp.float32,
                  precision=lax.Precision.HIGHEST)
        + b1[...]
    )
    h = jnp.maximum(pre, 0.0)
    y2[...] = jnp.dot(h, w2l[...], preferred_element_type=jnp.float32,
                      precision=lax.Precision.HIGHEST)
    z2[...] = jnp.dot(h, w2r[...], preferred_element_type=jnp.float32,
                      precision=lax.Precision.HIGHEST) + b2[...]


def _tc_final_body(agg0, agg1, z2, deg, out):
    recip = 1.0 / jnp.maximum(deg[:, :1], 1.0)
    out[...] = jnp.concatenate([agg0[...] * recip, agg1[...] * recip], axis=1) + z2[...]


_BM = 1000


def _row_spec(cols):
    return pl.BlockSpec((_BM, cols), lambda i: (i, 0))


def _full_spec(rows, cols):
    return pl.BlockSpec((rows, cols), lambda i: (0, 0))


def _tc_fused(agg0, agg1, x, deg, w1l, w1r, b1, w2l, w2r, b2):
    return pl.pallas_call(
        _tc_fused_body,
        grid=(N // _BM,),
        in_specs=[
            _row_spec(128), _row_spec(128), _row_spec(D_IN), _row_spec(128),
            _full_spec(D_IN, D_H), _full_spec(D_IN, D_H), _full_spec(1, D_H),
            _full_spec(D_H, D_OUT), _full_spec(D_H, D_OUT), _full_spec(1, D_OUT),
        ],
        out_specs=[_row_spec(D_OUT), _row_spec(D_OUT)],
        out_shape=[
            jax.ShapeDtypeStruct((N, D_OUT), jnp.float32),
            jax.ShapeDtypeStruct((N, D_OUT), jnp.float32),
        ],
    )(agg0, agg1, x, deg, w1l, w1r, b1, w2l, w2r, b2)


def _tc_final(agg0, agg1, z2, deg):
    return pl.pallas_call(
        _tc_final_body,
        grid=(N // _BM,),
        in_specs=[_row_spec(128), _row_spec(128), _row_spec(D_OUT), _row_spec(128)],
        out_specs=_row_spec(D_OUT),
        out_shape=jax.ShapeDtypeStruct((N, D_OUT), jnp.float32),
    )(agg0, agg1, z2, deg)


def kernel(x, edge_index, W1_l, b1, W1_r, W2_l, b2, W2_r):
    src = edge_index[0].astype(jnp.int32)
    dst = edge_index[1].astype(jnp.int32)
    pad = EPAD - E
    src_p = jnp.concatenate([src, jnp.zeros((pad,), jnp.int32)])
    dst_p = jnp.concatenate([dst, jnp.full((pad,), N, jnp.int32)])
    srcx = jnp.stack([2 * src_p, 2 * src_p + 1]).reshape(2, NS, CHUNKS, GB)
    dstx = dst_p.reshape(NS, CHUNKS, GB)

    zeros = jnp.zeros((GB, 128), jnp.float32)

    ones_table = jnp.concatenate(
        [jnp.ones((8, 128), jnp.float32), jnp.zeros((8, 128), jnp.float32)])
    srcx_deg = jnp.where(
        jnp.arange(EPAD, dtype=jnp.int32)[None, :] < E, 0, 8
    ).astype(jnp.int32).reshape(1, NS, CHUNKS, GB)
    srcx_deg = jnp.broadcast_to(srcx_deg, (2, NS, CHUNKS, GB))
    deg128, _unused = _sc_agg(ones_table, srcx_deg, dstx, zeros)

    table1 = x.reshape(2 * N, 128)
    agg0, agg1 = _sc_agg(table1, srcx, dstx, zeros)

    y2, z2 = _tc_fused(
        agg0, agg1, x, deg128,
        W1_l.T, W1_r.T, b1[None], W2_l.T, W2_r.T, b2[None],
    )

    table2 = y2.reshape(2 * N, 128)
    agg2_0, agg2_1 = _sc_agg(table2, srcx, dstx, zeros)

    return _tc_final(agg2_0, agg2_1, z2, deg128)


# speedup vs baseline: 8.8322x; 8.8322x over previous
import jax
import jax.numpy as jnp
from jax import lax
from jax.experimental import pallas as pl
from jax.experimental.pallas import tpu as pltpu
from jax.experimental.pallas import tpu_sc as plsc

N = 10000
E = 160000
D_IN = 256
D_H = 512
D_OUT = 256

NC = 2
NS = 16
GB = 64
IDXB = 8
CHUNKS = -(-E // (NS * GB * IDXB)) * IDXB
NB = CHUNKS // IDXB
EPAD = CHUNKS * NS * GB
RPT = -(-(N + 8) // (NS * 8)) * 8
NPAD = RPT * NS
TAIL = N - (NS - 1) * RPT


def _row_chunks(total):
    out = []
    off = 0
    while off < total:
        sz = min(GB, total - off)
        out.append((off, sz))
        off += sz
    return out


def _sc_agg_body(table, srcx, dstx, zeros_hbm, agg0_out, agg1_out,
                 idx_s, idx_d, gbuf, acc, sem):
    c = lax.axis_index("c")
    s = lax.axis_index("s")
    base = s * RPT

    pltpu.sync_copy(zeros_hbm, gbuf)
    for off, sz in _row_chunks(RPT):
        pltpu.sync_copy(gbuf.at[pl.ds(0, sz)], acc.at[pl.ds(base + off, sz)])
    plsc.subcore_barrier()

    def block(bi, _):
        pltpu.sync_copy(srcx.at[c, s, pl.ds(bi * IDXB, IDXB)], idx_s)
        pltpu.sync_copy(dstx.at[s, pl.ds(bi * IDXB, IDXB)], idx_d)

        def step(j, _):
            pltpu.async_copy(table.at[idx_s.at[j]], gbuf, sem).wait()
            pltpu.sync_copy(gbuf, acc.at[idx_d.at[j]], add=True)
            return _

        return lax.fori_loop(0, IDXB, step, _)

    lax.fori_loop(0, NB, block, None)
    plsc.subcore_barrier()

    for cid, out_ref in ((0, agg0_out), (1, agg1_out)):
        for last in (False, True):
            nrows = TAIL if last else RPT
            cond = jnp.logical_and(c == cid,
                                   (s == NS - 1) if last else (s != NS - 1))
            for off, sz in _row_chunks(nrows):
                @pl.when(cond)
                def _(off=off, sz=sz, out_ref=out_ref):
                    pltpu.sync_copy(acc.at[pl.ds(base + off, sz)],
                                    gbuf.at[pl.ds(0, sz)])
                    pltpu.sync_copy(gbuf.at[pl.ds(0, sz)],
                                    out_ref.at[pl.ds(base + off, sz)])


_SC_MESH = plsc.VectorSubcoreMesh(
    core_axis_name="c", subcore_axis_name="s", num_cores=NC, num_subcores=NS
)

_sc_agg = pl.kernel(
    _sc_agg_body,
    out_type=[
        jax.ShapeDtypeStruct((N, 128), jnp.float32),
        jax.ShapeDtypeStruct((N, 128), jnp.float32),
    ],
    mesh=_SC_MESH,
    scratch_types=[
        pltpu.VMEM((IDXB, GB), jnp.int32),
        pltpu.VMEM((IDXB, GB), jnp.int32),
        pltpu.VMEM((GB, 128), jnp.float32),
        pltpu.VMEM_SHARED((NPAD, 128), jnp.float32),
        pltpu.SemaphoreType.DMA,
    ],
)


def _tc_fused_body(agg0, agg1, x, deg, w1l, w1r, b1, w2l, w2r, b2, y2, z2):
    recip = 1.0 / jnp.maximum(deg[:, :1], 1.0)
    mean = jnp.concatenate([agg0[...] * recip, agg1[...] * recip], axis=1)
    pre = (
        jnp.dot(mean, w1l[...], preferred_element_type=jnp.float32,
                precision=lax.Precision.HIGHEST)
        + jnp.dot(x[...], w1r[...], preferred_element_type=jnp.float32,
                  precision=lax.Precision.HIGHEST)
        + b1[...]
    )
    h = jnp.maximum(pre, 0.0)
    y2[...] = jnp.dot(h, w2l[...], preferred_element_type=jnp.float32,
                      precision=lax.Precision.HIGHEST)
    z2[...] = jnp.dot(h, w2r[...], preferred_element_type=jnp.float32,
                      precision=lax.Precision.HIGHEST) + b2[...]


def _tc_final_body(agg0, agg1, z2, deg, out):
    recip = 1.0 / jnp.maximum(deg[:, :1], 1.0)
    out[...] = jnp.concatenate([agg0[...] * recip, agg1[...] * recip], axis=1) + z2[...]


_BM = 1000


def _row_spec(cols):
    return pl.BlockSpec((_BM, cols), lambda i: (i, 0))


def _full_spec(rows, cols):
    return pl.BlockSpec((rows, cols), lambda i: (0, 0))


def _tc_fused(agg0, agg1, x, deg, w1l, w1r, b1, w2l, w2r, b2):
    return pl.pallas_call(
        _tc_fused_body,
        grid=(N // _BM,),
        in_specs=[
            _row_spec(128), _row_spec(128), _row_spec(D_IN), _row_spec(128),
            _full_spec(D_IN, D_H), _full_spec(D_IN, D_H), _full_spec(1, D_H),
            _full_spec(D_H, D_OUT), _full_spec(D_H, D_OUT), _full_spec(1, D_OUT),
        ],
        out_specs=[_row_spec(D_OUT), _row_spec(D_OUT)],
        out_shape=[
            jax.ShapeDtypeStruct((N, D_OUT), jnp.float32),
            jax.ShapeDtypeStruct((N, D_OUT), jnp.float32),
        ],
    )(agg0, agg1, x, deg, w1l, w1r, b1, w2l, w2r, b2)


def _tc_final(agg0, agg1, z2, deg):
    return pl.pallas_call(
        _tc_final_body,
        grid=(N // _BM,),
        in_specs=[_row_spec(128), _row_spec(128), _row_spec(D_OUT), _row_spec(128)],
        out_specs=_row_spec(D_OUT),
        out_shape=jax.ShapeDtypeStruct((N, D_OUT), jnp.float32),
    )(agg0, agg1, z2, deg)


def kernel(x, edge_index, W1_l, b1, W1_r, W2_l, b2, W2_r):
    src = edge_index[0].astype(jnp.int32)
    dst = edge_index[1].astype(jnp.int32)
    pad = EPAD - E
    src_p = jnp.concatenate([src, jnp.zeros((pad,), jnp.int32)])
    dst_p = jnp.concatenate([dst, jnp.full((pad,), N, jnp.int32)])
    srcx = jnp.stack([2 * src_p, 2 * src_p + 1]).reshape(2, NS, CHUNKS, GB)
    dstx = dst_p.reshape(NS, CHUNKS, GB)

    zeros = jnp.zeros((GB, 128), jnp.float32)

    ones_table = jnp.concatenate(
        [jnp.ones((2048, 128), jnp.float32), jnp.zeros((8, 128), jnp.float32)])
    e_iota = jnp.arange(EPAD, dtype=jnp.int32)
    srcx_deg = jnp.where(e_iota < E, e_iota % 2048, 2048)[None, :]
    srcx_deg = jnp.broadcast_to(
        srcx_deg.reshape(1, NS, CHUNKS, GB), (2, NS, CHUNKS, GB))
    deg128, _unused = _sc_agg(ones_table, srcx_deg, dstx, zeros)

    table1 = x.reshape(2 * N, 128)
    agg0, agg1 = _sc_agg(table1, srcx, dstx, zeros)

    y2, z2 = _tc_fused(
        agg0, agg1, x, deg128,
        W1_l.T, W1_r.T, b1[None], W2_l.T, W2_r.T, b2[None],
    )

    table2 = y2.reshape(2 * N, 128)
    agg2_0, agg2_1 = _sc_agg(table2, srcx, dstx, zeros)

    return _tc_final(agg2_0, agg2_1, z2, deg128)


# speedup vs baseline: 13.2000x; 1.4945x over previous
import jax
import jax.numpy as jnp
from jax import lax
from jax.experimental import pallas as pl
from jax.experimental.pallas import tpu as pltpu
from jax.experimental.pallas import tpu_sc as plsc

N = 10000
E = 160000
D_IN = 256
D_H = 512
D_OUT = 256

NC = 2
NS = 16
GB = 64
IDXB = 8
CHUNKS = -(-E // (NS * GB * IDXB)) * IDXB
NB = CHUNKS // IDXB
EPAD = CHUNKS * NS * GB
RPT = -(-(N + 8) // (NS * 8)) * 8
NPAD = RPT * NS
TAIL = N - (NS - 1) * RPT


def _row_chunks(total):
    out = []
    off = 0
    while off < total:
        sz = min(GB, total - off)
        out.append((off, sz))
        off += sz
    return out


def _zero_acc(zeros_hbm, stage, acc, base):
    pltpu.sync_copy(zeros_hbm, stage)
    for off, sz in _row_chunks(RPT):
        pltpu.sync_copy(stage.at[pl.ds(0, sz)], acc.at[pl.ds(base + off, sz)])


def _write_out(c, s, base, stage, acc, out_refs):
    for cid, out_ref in out_refs:
        for last in (False, True):
            nrows = TAIL if last else RPT
            cond = jnp.logical_and(c == cid,
                                   (s == NS - 1) if last else (s != NS - 1))
            for off, sz in _row_chunks(nrows):
                @pl.when(cond)
                def _(off=off, sz=sz, out_ref=out_ref):
                    pltpu.sync_copy(acc.at[pl.ds(base + off, sz)],
                                    stage.at[pl.ds(0, sz)])
                    pltpu.sync_copy(stage.at[pl.ds(0, sz)],
                                    out_ref.at[pl.ds(base + off, sz)])


def _sc_agg_body(table, srcx, dstx, zeros_hbm, agg0_out, agg1_out,
                 idx_s, idx_d, g0, g1, acc, sem):
    c = lax.axis_index("c")
    s = lax.axis_index("s")
    base = s * RPT
    _zero_acc(zeros_hbm, g0, acc, base)
    plsc.subcore_barrier()

    bufs = (g0, g1)

    def block(bi, _):
        pltpu.sync_copy(srcx.at[c, s, pl.ds(bi * IDXB, IDXB)], idx_s)
        pltpu.sync_copy(dstx.at[s, pl.ds(bi * IDXB, IDXB)], idx_d)
        d = pltpu.async_copy(table.at[idx_s.at[0]], bufs[0], sem)
        for j in range(IDXB):
            d.wait()
            if j + 1 < IDXB:
                d = pltpu.async_copy(table.at[idx_s.at[j + 1]],
                                     bufs[(j + 1) % 2], sem)
            pltpu.sync_copy(bufs[j % 2], acc.at[idx_d.at[j]], add=True)
        return _

    lax.fori_loop(0, NB, block, None)
    plsc.subcore_barrier()
    _write_out(c, s, base, g0, acc, ((0, agg0_out), (1, agg1_out)))


def _sc_deg_body(ones_hbm, dstx, zeros_hbm, d0_out, d1_out,
                 idx_d, ones_v, acc, sem):
    c = lax.axis_index("c")
    s = lax.axis_index("s")
    base = s * RPT
    _zero_acc(zeros_hbm, ones_v, acc, base)
    pltpu.sync_copy(ones_hbm, ones_v)
    plsc.subcore_barrier()

    def block(bi, _):
        bij = bi + c * (NB // 2)
        pltpu.sync_copy(dstx.at[s, pl.ds(bij * IDXB, IDXB)], idx_d)

        def step(j, _):
            pltpu.sync_copy(ones_v, acc.at[idx_d.at[j]], add=True)
            return _

        return lax.fori_loop(0, IDXB, step, _)

    lax.fori_loop(0, NB // 2, block, None)
    plsc.subcore_barrier()
    _write_out(c, s, base, ones_v, acc, ((0, d0_out), (1, d1_out)))


_SC_MESH = plsc.VectorSubcoreMesh(
    core_axis_name="c", subcore_axis_name="s", num_cores=NC, num_subcores=NS
)

_AGG_OUT = [
    jax.ShapeDtypeStruct((N, 128), jnp.float32),
    jax.ShapeDtypeStruct((N, 128), jnp.float32),
]

_sc_agg = pl.kernel(
    _sc_agg_body,
    out_type=_AGG_OUT,
    mesh=_SC_MESH,
    scratch_types=[
        pltpu.VMEM((IDXB, GB), jnp.int32),
        pltpu.VMEM((IDXB, GB), jnp.int32),
        pltpu.VMEM((GB, 128), jnp.float32),
        pltpu.VMEM((GB, 128), jnp.float32),
        pltpu.VMEM_SHARED((NPAD, 128), jnp.float32),
        pltpu.SemaphoreType.DMA,
    ],
)

_sc_deg = pl.kernel(
    _sc_deg_body,
    out_type=_AGG_OUT,
    mesh=_SC_MESH,
    scratch_types=[
        pltpu.VMEM((IDXB, GB), jnp.int32),
        pltpu.VMEM((GB, 128), jnp.float32),
        pltpu.VMEM_SHARED((NPAD, 128), jnp.float32),
        pltpu.SemaphoreType.DMA,
    ],
)


def _tc_fused_body(agg0, agg1, x, d0, d1, w1l, w1r, b1, w2l, w2r, b2, y2, z2):
    recip = 1.0 / jnp.maximum(d0[:, :1] + d1[:, :1], 1.0)
    mean = jnp.concatenate([agg0[...] * recip, agg1[...] * recip], axis=1)
    pre = (
        jnp.dot(mean, w1l[...], preferred_element_type=jnp.float32,
                precision=lax.Precision.HIGHEST)
        + jnp.dot(x[...], w1r[...], preferred_element_type=jnp.float32,
                  precision=lax.Precision.HIGHEST)
        + b1[...]
    )
    h = jnp.maximum(pre, 0.0)
    y2[...] = jnp.dot(h, w2l[...], preferred_element_type=jnp.float32,
                      precision=lax.Precision.HIGHEST)
    z2[...] = jnp.dot(h, w2r[...], preferred_element_type=jnp.float32,
                      precision=lax.Precision.HIGHEST) + b2[...]


def _tc_final_body(agg0, agg1, z2, d0, d1, out):
    recip = 1.0 / jnp.maximum(d0[:, :1] + d1[:, :1], 1.0)
    out[...] = jnp.concatenate([agg0[...] * recip, agg1[...] * recip], axis=1) + z2[...]


_BM = 1000


def _row_spec(cols):
    return pl.BlockSpec((_BM, cols), lambda i: (i, 0))


def _full_spec(rows, cols):
    return pl.BlockSpec((rows, cols), lambda i: (0, 0))


def _tc_fused(agg0, agg1, x, d0, d1, w1l, w1r, b1, w2l, w2r, b2):
    return pl.pallas_call(
        _tc_fused_body,
        grid=(N // _BM,),
        in_specs=[
            _row_spec(128), _row_spec(128), _row_spec(D_IN),
            _row_spec(128), _row_spec(128),
            _full_spec(D_IN, D_H), _full_spec(D_IN, D_H), _full_spec(1, D_H),
            _full_spec(D_H, D_OUT), _full_spec(D_H, D_OUT), _full_spec(1, D_OUT),
        ],
        out_specs=[_row_spec(D_OUT), _row_spec(D_OUT)],
        out_shape=[
            jax.ShapeDtypeStruct((N, D_OUT), jnp.float32),
            jax.ShapeDtypeStruct((N, D_OUT), jnp.float32),
        ],
    )(agg0, agg1, x, d0, d1, w1l, w1r, b1, w2l, w2r, b2)


def _tc_final(agg0, agg1, z2, d0, d1):
    return pl.pallas_call(
        _tc_final_body,
        grid=(N // _BM,),
        in_specs=[_row_spec(128), _row_spec(128), _row_spec(D_OUT),
                  _row_spec(128), _row_spec(128)],
        out_specs=_row_spec(D_OUT),
        out_shape=jax.ShapeDtypeStruct((N, D_OUT), jnp.float32),
    )(agg0, agg1, z2, d0, d1)


def kernel(x, edge_index, W1_l, b1, W1_r, W2_l, b2, W2_r):
    src = edge_index[0].astype(jnp.int32)
    dst = edge_index[1].astype(jnp.int32)
    pad = EPAD - E
    src_p = jnp.concatenate([src, jnp.zeros((pad,), jnp.int32)])
    dst_p = jnp.concatenate([dst, jnp.full((pad,), N, jnp.int32)])
    srcx = jnp.stack([2 * src_p, 2 * src_p + 1]).reshape(2, NS, CHUNKS, GB)
    dstx = dst_p.reshape(NS, CHUNKS, GB)

    zeros = jnp.zeros((GB, 128), jnp.float32)

    ones_gb = jnp.ones((GB, 128), jnp.float32)
    d0, d1 = _sc_deg(ones_gb, dstx, zeros)

    table1 = x.reshape(2 * N, 128)
    agg0, agg1 = _sc_agg(table1, srcx, dstx, zeros)

    y2, z2 = _tc_fused(
        agg0, agg1, x, d0, d1,
        W1_l.T, W1_r.T, b1[None], W2_l.T, W2_r.T, b2[None],
    )

    table2 = y2.reshape(2 * N, 128)
    agg2_0, agg2_1 = _sc_agg(table2, srcx, dstx, zeros)

    return _tc_final(agg2_0, agg2_1, z2, d0, d1)
